# Initial kernel scaffold; baseline (speedup 1.0000x reference)
import jax
import jax.numpy as jnp
from jax.experimental import pallas as pl


def kernel(features, coors, batch_size, W0, W1):
    raise NotImplementedError("write your pallas kernel here")


# speedup vs baseline: 118.9944x; 118.9944x over previous
import functools

import jax
import jax.numpy as jnp
from jax import lax
from jax.experimental import pallas as pl
from jax.experimental.pallas import tpu as pltpu
from jax.experimental.pallas import tpu_sc as plsc

N = 150000
C = 16
D, H, W = 21, 800, 704
B = 2
K = 27
TAB = B * D * H * W

NC = 2
NS = 16
NW = NC * NS
CH = 4736
NPAD = NW * CH
HALF = CH // 2
SENT_BASE = N
SENT_MASK = 1023

_mesh = plsc.VectorSubcoreMesh(core_axis_name="c", subcore_axis_name="s")


def _wid():
    return lax.axis_index("s") * NC + lax.axis_index("c")


_OFFS = [(dz, dy, dx)
         for dz in (-1, 0, 1) for dy in (-1, 0, 1) for dx in (-1, 0, 1)]


@functools.partial(
    pl.kernel,
    out_type=jax.ShapeDtypeStruct((K * NPAD,), jnp.int32),
    mesh=_mesh,
    scratch_types=[
        pltpu.VMEM((CH,), jnp.int32),
        pltpu.VMEM((CH,), jnp.int32),
        pltpu.VMEM((CH,), jnp.int32),
        pltpu.VMEM((CH,), jnp.int32),
        pltpu.VMEM((CH,), jnp.int32),
        pltpu.VMEM((CH,), jnp.int32),
        pltpu.VMEM((CH,), jnp.int32),
        pltpu.VMEM((CH,), jnp.int32),
        pltpu.SemaphoreType.DMA,
    ],
    compiler_params=pltpu.CompilerParams(use_tc_tiling_on_sc=False),
)
def _rulebook(b_h, z_h, y_h, x_h, table_h, rb_h,
              zb, yb, xb, lb, nb, vb, tb, ob, sem):
    base = _wid() * CH
    pltpu.sync_copy(z_h.at[pl.ds(base, CH)], zb)
    pltpu.sync_copy(y_h.at[pl.ds(base, CH)], yb)
    pltpu.sync_copy(x_h.at[pl.ds(base, CH)], xb)
    pltpu.sync_copy(b_h.at[pl.ds(base, CH)], lb)

    def init(j, _):
        s = j * 16
        lv = ((lb[pl.ds(s, 16)] * D + zb[pl.ds(s, 16)]) * H
              + yb[pl.ds(s, 16)]) * W + xb[pl.ds(s, 16)]
        lb[pl.ds(s, 16)] = lv
        return 0
    lax.fori_loop(0, CH // 16, init, 0)

    def per_k(k, _):
        dz = k // 9 - 1
        dy = (k // 3) % 3 - 1
        dx = k % 3 - 1
        dlin = (dz * H + dy) * W + dx

        def scan1(j, _):
            s = j * 16
            nz = zb[pl.ds(s, 16)] + dz
            ny = yb[pl.ds(s, 16)] + dy
            nx = xb[pl.ds(s, 16)] + dx
            ok = ((nz >= 0) & (nz < D) & (ny >= 0) & (ny < H)
                  & (nx >= 0) & (nx < W))
            iot = lax.broadcasted_iota(jnp.int32, (16,), 0)
            spread = (base + s + iot) & 0xFFFF
            nb[pl.ds(s, 16)] = jnp.where(ok, lb[pl.ds(s, 16)] + dlin, spread)
            vb[pl.ds(s, 16)] = jnp.where(ok, 1, 0).astype(jnp.int32)
            return 0
        lax.fori_loop(0, CH // 16, scan1, 0)

        pltpu.async_copy(table_h.at[nb], tb, sem).wait()

        def scan2(j, _):
            s = j * 16
            nidx = tb[pl.ds(s, 16)]
            ok = (vb[pl.ds(s, 16)] > 0) & (nidx >= 0)
            iot = lax.broadcasted_iota(jnp.int32, (16,), 0)
            snt = SENT_BASE + ((base + s + iot) & SENT_MASK)
            ob[pl.ds(s, 16)] = jnp.where(ok, nidx, snt) * K + k
            return 0
        lax.fori_loop(0, CH // 16, scan2, 0)

        pltpu.sync_copy(ob, rb_h.at[pl.ds(k * NPAD + base, CH)])
        return 0
    lax.fori_loop(0, K, per_k, 0)


@functools.partial(
    pl.kernel,
    out_type=jax.ShapeDtypeStruct((NPAD, C), jnp.float32),
    mesh=_mesh,
    scratch_types=[
        pltpu.VMEM((HALF,), jnp.int32),
        pltpu.VMEM((HALF, C), jnp.float32),
        pltpu.VMEM((HALF, C), jnp.float32),
        pltpu.SemaphoreType.DMA,
    ],
    compiler_params=pltpu.CompilerParams(use_tc_tiling_on_sc=False),
)
def _accum(rb_h, y2_h, out_h, idxv, buf, acc, sem):
    for half in range(2):
        base = _wid() * CH + half * HALF

        pltpu.sync_copy(rb_h.at[pl.ds(base, HALF)], idxv)
        pltpu.async_copy(y2_h.at[idxv], buf, sem).wait()

        def strow(j, _):
            for u in range(8):
                r = j * 8 + u
                acc[r] = buf[r]
            return 0
        lax.fori_loop(0, HALF // 8, strow, 0)

        def per_k(k, _):
            pltpu.sync_copy(rb_h.at[pl.ds(k * NPAD + base, HALF)], idxv)
            pltpu.async_copy(y2_h.at[idxv], buf, sem).wait()

            def addrow(j, _):
                for u in range(8):
                    r = j * 8 + u
                    plsc.addupdate(acc.at[r], buf[r])
                return 0
            lax.fori_loop(0, HALF // 8, addrow, 0)
            return 0
        lax.fori_loop(1, K, per_k, 0)

        pltpu.sync_copy(acc, out_h.at[pl.ds(base, HALF)])


_MMBLK = 1024


def _mm_body(x_ref, w_ref, o_ref):
    o_ref[...] = jnp.dot(x_ref[...], w_ref[...],
                         preferred_element_type=jnp.float32)


def _matmul(x, w):
    return pl.pallas_call(
        _mm_body,
        grid=(NPAD // _MMBLK,),
        in_specs=[pl.BlockSpec((_MMBLK, C), lambda i: (i, 0)),
                  pl.BlockSpec((C, K * C), lambda i: (0, 0))],
        out_specs=pl.BlockSpec((_MMBLK, K * C), lambda i: (i, 0)),
        out_shape=jax.ShapeDtypeStruct((NPAD, K * C), jnp.float32),
    )(x, w)


def kernel(features, coors, batch_size, W0, W1):
    off = jnp.asarray(batch_size, jnp.int32) - jnp.int32(B)
    bcol = coors[:, 0]
    zcol = coors[:, 1]
    ycol = coors[:, 2]
    xcol = coors[:, 3]
    lin = ((bcol * D + zcol) * H + ycol) * W + xcol + off
    table = jnp.full((TAB,), -1, jnp.int32).at[lin].max(
        jnp.arange(N, dtype=jnp.int32))

    pad = NPAD - N
    bp = jnp.concatenate([bcol, jnp.zeros((pad,), jnp.int32)])
    zp = jnp.concatenate([zcol, jnp.full((pad,), -10000, jnp.int32)])
    yp = jnp.concatenate([ycol, jnp.full((pad,), -10000, jnp.int32)])
    xp = jnp.concatenate([xcol, jnp.full((pad,), -10000, jnp.int32)])
    featp = jnp.concatenate(
        [features, jnp.zeros((pad, C), jnp.float32)])

    rb = _rulebook(bp, zp, yp, xp, table)

    wc0 = W0.transpose(1, 0, 2).reshape(C, K * C)
    wc1 = W1.transpose(1, 0, 2).reshape(C, K * C)

    y = _matmul(featp, wc0).reshape(NPAD * K, C)
    h = _accum(rb, y)
    y = _matmul(h, wc1).reshape(NPAD * K, C)
    h = _accum(rb, y)
    return h[:N]


# speedup vs baseline: 275.8701x; 2.3183x over previous
import functools

import jax
import jax.numpy as jnp
from jax import lax
from jax.experimental import pallas as pl
from jax.experimental.pallas import tpu as pltpu
from jax.experimental.pallas import tpu_sc as plsc

N = 150000
C = 16
D, H, W = 21, 800, 704
B = 2
K = 27
TAB = B * D * H * W

NC = 2
NS = 16
NW = NC * NS
CH = 4736
NPAD = NW * CH
SENT_BASE = N
SENT_MASK = 1023

PBUF = 16384
FLUSH_T = PBUF - 2 * CH - 16
PCAP = 512 * 260
CHUNK = 512
NKW = 28

_mesh = plsc.VectorSubcoreMesh(core_axis_name="c", subcore_axis_name="s")


def _wid():
    return lax.axis_index("s") * NC + lax.axis_index("c")


def _iota16():
    return lax.broadcasted_iota(jnp.int32, (16,), 0)


@functools.partial(
    pl.kernel,
    out_type=(
        jax.ShapeDtypeStruct((NW * PCAP,), jnp.int32),
        jax.ShapeDtypeStruct((NW * PCAP,), jnp.int32),
        jax.ShapeDtypeStruct((NW * 16,), jnp.int32),
        jax.ShapeDtypeStruct((NPAD,), jnp.int32),
    ),
    mesh=_mesh,
    scratch_types=[
        pltpu.VMEM((CH,), jnp.int32),
        pltpu.VMEM((CH,), jnp.int32),
        pltpu.VMEM((CH,), jnp.int32),
        pltpu.VMEM((CH,), jnp.int32),
        pltpu.VMEM((CH,), jnp.int32),
        pltpu.VMEM((CH,), jnp.int32),
        pltpu.VMEM((CH,), jnp.int32),
        pltpu.VMEM((CH,), jnp.int32),
        pltpu.VMEM((CH,), jnp.int32),
        pltpu.VMEM((CH,), jnp.int32),
        pltpu.VMEM((PBUF,), jnp.int32),
        pltpu.VMEM((PBUF,), jnp.int32),
        pltpu.VMEM((16,), jnp.int32),
        pltpu.VMEM((CH,), jnp.int32),
        pltpu.SemaphoreType.DMA,
        pltpu.SemaphoreType.DMA,
    ],
    compiler_params=pltpu.CompilerParams(use_tc_tiling_on_sc=False,
                                         needs_layout_passes=False),
)
def _rulebook(b_h, z_h, y_h, x_h, table_h, pn_h, pm_h, cnt_h, cidx_h,
              zb, yb, xb, lb, nb0, nb1, vb0, vb1, tb0, tb1,
              pn, pm, cb, cvb, gs0, gs1):
    base = _wid() * CH
    pbase = _wid() * PCAP
    pltpu.sync_copy(z_h.at[pl.ds(base, CH)], zb)
    pltpu.sync_copy(y_h.at[pl.ds(base, CH)], yb)
    pltpu.sync_copy(x_h.at[pl.ds(base, CH)], xb)
    pltpu.sync_copy(b_h.at[pl.ds(base, CH)], lb)

    def init(j, _):
        s = j * 16
        lv = ((lb[pl.ds(s, 16)] * D + zb[pl.ds(s, 16)]) * H
              + yb[pl.ds(s, 16)]) * W + xb[pl.ds(s, 16)]
        lb[pl.ds(s, 16)] = lv
        return 0
    lax.fori_loop(0, CH // 16, init, 0)

    def scan1(k, nb, vb):
        dz = k // 9 - 1
        dy = (k // 3) % 3 - 1
        dx = k % 3 - 1
        dlin = (dz * H + dy) * W + dx

        def body(j, _):
            s = j * 16
            nz = zb[pl.ds(s, 16)] + dz
            ny = yb[pl.ds(s, 16)] + dy
            nx = xb[pl.ds(s, 16)] + dx
            ok = ((nz >= 0) & (nz < D) & (ny >= 0) & (ny < H)
                  & (nx >= 0) & (nx < W))
            spread = (base + s + _iota16()) & 0xFFFF
            nb[pl.ds(s, 16)] = jnp.where(ok, lb[pl.ds(s, 16)] + dlin, spread)
            vb[pl.ds(s, 16)] = jnp.where(ok, 1, 0)
            return 0
        lax.fori_loop(0, CH // 16, body, 0)

    def compact(k, vb, tb, off):
        def center(off):
            def body(j, _):
                s = j * 16
                nidx = tb[pl.ds(s, 16)]
                ok = (vb[pl.ds(s, 16)] > 0) & (nidx >= 0)
                snt = SENT_BASE + ((base + s + _iota16()) & SENT_MASK)
                cvb[pl.ds(s, 16)] = jnp.where(ok, nidx, snt)
                return 0
            lax.fori_loop(0, CH // 16, body, 0)
            pltpu.sync_copy(cvb, cidx_h.at[pl.ds(base, CH)])
            return off

        def taps(off):
            def body(j, off):
                s = j * 16
                nidx = tb[pl.ds(s, 16)]
                li = s + _iota16()
                ok = (vb[pl.ds(s, 16)] > 0) & (nidx >= 0)
                cs1 = plsc.cumsum(jnp.where(ok, 1, 0))
                pos1 = off - 1 + cs1
                plsc.store_scatter(pn, [pos1], nidx, mask=ok)
                plsc.store_scatter(pm, [pos1], li * 32 + k, mask=ok)
                return off + cs1[15]
            return lax.fori_loop(0, CH // 16, body, off)
        return lax.cond(k == 13, center, taps, off)

    def flush(off, flushed):
        nch = off // CHUNK
        flushed = pl.multiple_of(flushed, 8)

        def fl(j, _):
            pltpu.sync_copy(pn.at[pl.ds(j * CHUNK, CHUNK)],
                            pn_h.at[pl.ds(pbase + flushed + j * CHUNK,
                                          CHUNK)])
            pltpu.sync_copy(pm.at[pl.ds(j * CHUNK, CHUNK)],
                            pm_h.at[pl.ds(pbase + flushed + j * CHUNK,
                                          CHUNK)])
            return 0
        lax.fori_loop(0, nch, fl, 0)
        rem = off - nch * CHUNK

        nb = pl.multiple_of(nch * CHUNK, 8)

        def mv(j, _):
            s = j * 16
            pn[pl.ds(s, 16)] = pn[pl.ds(nb + s, 16)]
            pm[pl.ds(s, 16)] = pm[pl.ds(nb + s, 16)]
            return 0
        lax.fori_loop(0, (rem + 15) // 16, mv, 0)
        return rem, flushed + nch * CHUNK

    def maybe_flush(off, flushed):
        return lax.cond(off >= FLUSH_T, flush,
                        lambda o, f: (o, f), off, flushed)

    scan1(0, nb0, vb0)
    pltpu.async_copy(table_h.at[nb0], tb0, gs0)

    def pair_loop(j, carry):
        off, flushed = carry
        k0 = 2 * j + 1
        k1 = 2 * j + 2
        scan1(k0, nb1, vb1)
        pltpu.async_copy(table_h.at[nb1], tb1, gs1)

        pltpu.make_async_copy(table_h.at[nb0], tb0, gs0).wait()
        off = compact(2 * j, vb0, tb0, off)
        off, flushed = maybe_flush(off, flushed)

        scan1(k1, nb0, vb0)
        pltpu.async_copy(table_h.at[nb0], tb0, gs0)

        pltpu.make_async_copy(table_h.at[nb1], tb1, gs1).wait()
        off = compact(k0, vb1, tb1, off)
        off, flushed = maybe_flush(off, flushed)
        return off, flushed
    off, flushed = lax.fori_loop(0, 13, pair_loop, (0, 0))

    pltpu.make_async_copy(table_h.at[nb0], tb0, gs0).wait()
    off = compact(26, vb0, tb0, off)

    cnt = flushed + off
    end = ((off + CHUNK - 1) // CHUNK) * CHUNK

    def padw(j, _):
        pos = off + j * 16 + _iota16()
        snt = SENT_BASE + (pos & SENT_MASK)
        m = pos < end
        plsc.store_scatter(pn, [pos], snt, mask=m)
        plsc.store_scatter(pm, [pos], jnp.zeros((16,), jnp.int32), mask=m)
        return 0
    lax.fori_loop(0, (end - off + 15) // 16, padw, 0)

    flushed8 = pl.multiple_of(flushed, 8)

    def fl2(j, _):
        pltpu.sync_copy(pn.at[pl.ds(j * CHUNK, CHUNK)],
                        pn_h.at[pl.ds(pbase + flushed8 + j * CHUNK, CHUNK)])
        pltpu.sync_copy(pm.at[pl.ds(j * CHUNK, CHUNK)],
                        pm_h.at[pl.ds(pbase + flushed8 + j * CHUNK, CHUNK)])
        return 0
    lax.fori_loop(0, end // CHUNK, fl2, 0)

    cb[pl.ds(0, 16)] = jnp.zeros((16,), jnp.int32) + cnt
    pltpu.sync_copy(cb, cnt_h.at[pl.ds(_wid() * 16, 16)])


@functools.partial(
    pl.kernel,
    out_type=jax.ShapeDtypeStruct((NPAD, C), jnp.float32),
    mesh=_mesh,
    scratch_types=[
        pltpu.VMEM((16,), jnp.int32),
        pltpu.VMEM((NKW * C, C), jnp.float32),
        pltpu.VMEM((CHUNK,), jnp.int32),
        pltpu.VMEM((CHUNK,), jnp.int32),
        pltpu.VMEM((CHUNK, C), jnp.float32),
        pltpu.VMEM((CH, C), jnp.float32),
        pltpu.VMEM((CH,), jnp.int32),
        pltpu.SemaphoreType.DMA,
    ],
    compiler_params=pltpu.CompilerParams(use_tc_tiling_on_sc=False,
                                         needs_layout_passes=False),
)
def _apply(pn_h, pm_h, cnt_h, cidx_h, src_h, y13_h, wext_h, out_h,
           cntv, wv, iv, mv, gbuf, acc, civ, sem):
    base = _wid() * CH
    pbase = _wid() * PCAP
    pltpu.sync_copy(wext_h, wv)
    pltpu.sync_copy(cnt_h.at[pl.ds(_wid() * 16, 16)], cntv)
    pltpu.sync_copy(cidx_h.at[pl.ds(base, CH)], civ)
    pltpu.async_copy(y13_h.at[civ], acc, sem).wait()
    cnt = cntv[pl.ds(0, 16)][0]
    nch = (cnt + CHUNK - 1) // CHUNK

    def chunk(ci, _):
        pltpu.sync_copy(pn_h.at[pl.ds(pbase + ci * CHUNK, CHUNK)], iv)
        pltpu.sync_copy(pm_h.at[pl.ds(pbase + ci * CHUNK, CHUNK)], mv)
        pltpu.async_copy(src_h.at[iv], gbuf, sem).wait()

        def group(g, _):
            mvec = mv[pl.ds(g * 16, 16)]
            for p in range(16):
                m = mvec[p]
                li = m >> 5
                kk = (m & 31) * C
                row = gbuf[g * 16 + p]
                v = row[0] * wv[kk]
                for c in range(1, C):
                    v = v + row[c] * wv[kk + c]
                plsc.addupdate(acc.at[li], v)
            return 0
        lax.fori_loop(0, CHUNK // 16, group, 0)
        return 0
    lax.fori_loop(0, nch, chunk, 0)

    pltpu.sync_copy(acc, out_h.at[pl.ds(base, CH)])


_MMBLK = 4096


def _mm_body(x_ref, w_ref, o_ref):
    o_ref[...] = jnp.dot(x_ref[...], w_ref[...],
                         preferred_element_type=jnp.float32)


def _center_mm(x, w):
    return pl.pallas_call(
        _mm_body,
        grid=(NPAD // _MMBLK,),
        in_specs=[pl.BlockSpec((_MMBLK, C), lambda i: (i, 0)),
                  pl.BlockSpec((C, C), lambda i: (0, 0))],
        out_specs=pl.BlockSpec((_MMBLK, C), lambda i: (i, 0)),
        out_shape=jax.ShapeDtypeStruct((NPAD, C), jnp.float32),
    )(x, w)


def kernel(features, coors, batch_size, W0, W1):
    off = jnp.asarray(batch_size, jnp.int32) - jnp.int32(B)
    bcol = coors[:, 0]
    zcol = coors[:, 1]
    ycol = coors[:, 2]
    xcol = coors[:, 3]
    lin = ((bcol * D + zcol) * H + ycol) * W + xcol + off
    table = jnp.full((TAB,), -1, jnp.int32).at[lin].max(
        jnp.arange(N, dtype=jnp.int32))

    pad = NPAD - N
    bp = jnp.concatenate([bcol, jnp.zeros((pad,), jnp.int32)])
    zp = jnp.concatenate([zcol, jnp.full((pad,), -10000, jnp.int32)])
    yp = jnp.concatenate([ycol, jnp.full((pad,), -10000, jnp.int32)])
    xp = jnp.concatenate([xcol, jnp.full((pad,), -10000, jnp.int32)])
    featp = jnp.concatenate(
        [features, jnp.zeros((pad, C), jnp.float32)])

    pn, pm, cnt, cidx = _rulebook(bp, zp, yp, xp, table)

    wx0 = jnp.concatenate([W0, -W0[13:14]], axis=0).reshape(NKW * C, C)
    wx1 = jnp.concatenate([W1, -W1[13:14]], axis=0).reshape(NKW * C, C)

    y = _center_mm(featp, W0[13])
    h = _apply(pn, pm, cnt, cidx, featp, y, wx0)
    y = _center_mm(h, W1[13])
    h = _apply(pn, pm, cnt, cidx, h, y, wx1)
    return h[:N]


# speedup vs baseline: 291.0432x; 1.0550x over previous
import functools

import jax
import jax.numpy as jnp
from jax import lax
from jax.experimental import pallas as pl
from jax.experimental.pallas import tpu as pltpu
from jax.experimental.pallas import tpu_sc as plsc

N = 150000
C = 16
D, H, W = 21, 800, 704
B = 2
K = 27
TAB = B * D * H * W

NC = 2
NS = 16
NW = NC * NS
CH = 4736
NPAD = NW * CH
SENT_BASE = N
SENT_MASK = 1023

PBUF = 16384
FLUSH_T = PBUF - 2 * CH - 16
PCAP = 512 * 260
CHUNK = 512
NKW = 28

_mesh = plsc.VectorSubcoreMesh(core_axis_name="c", subcore_axis_name="s")


def _wid():
    return lax.axis_index("s") * NC + lax.axis_index("c")


def _iota16():
    return lax.broadcasted_iota(jnp.int32, (16,), 0)


@functools.partial(
    pl.kernel,
    out_type=(
        jax.ShapeDtypeStruct((NW * PCAP,), jnp.int32),
        jax.ShapeDtypeStruct((NW * PCAP,), jnp.int32),
        jax.ShapeDtypeStruct((NW * 16,), jnp.int32),
        jax.ShapeDtypeStruct((NPAD,), jnp.int32),
    ),
    mesh=_mesh,
    scratch_types=[
        pltpu.VMEM((CH,), jnp.int32),
        pltpu.VMEM((CH,), jnp.int32),
        pltpu.VMEM((CH,), jnp.int32),
        pltpu.VMEM((CH,), jnp.int32),
        pltpu.VMEM((CH,), jnp.int32),
        pltpu.VMEM((CH,), jnp.int32),
        pltpu.VMEM((CH,), jnp.int32),
        pltpu.VMEM((CH,), jnp.int32),
        pltpu.VMEM((CH,), jnp.int32),
        pltpu.VMEM((CH,), jnp.int32),
        pltpu.VMEM((PBUF,), jnp.int32),
        pltpu.VMEM((PBUF,), jnp.int32),
        pltpu.VMEM((16,), jnp.int32),
        pltpu.VMEM((CH,), jnp.int32),
        pltpu.SemaphoreType.DMA,
        pltpu.SemaphoreType.DMA,
    ],
    compiler_params=pltpu.CompilerParams(use_tc_tiling_on_sc=False,
                                         needs_layout_passes=False),
)
def _rulebook(b_h, z_h, y_h, x_h, table_h, pn_h, pm_h, cnt_h, cidx_h,
              zb, yb, xb, lb, nb0, nb1, vb0, vb1, tb0, tb1,
              pn, pm, cb, cvb, gs0, gs1):
    base = _wid() * CH
    pbase = _wid() * PCAP
    pltpu.sync_copy(z_h.at[pl.ds(base, CH)], zb)
    pltpu.sync_copy(y_h.at[pl.ds(base, CH)], yb)
    pltpu.sync_copy(x_h.at[pl.ds(base, CH)], xb)
    pltpu.sync_copy(b_h.at[pl.ds(base, CH)], lb)

    def init(j, _):
        s = j * 16
        lv = ((lb[pl.ds(s, 16)] * D + zb[pl.ds(s, 16)]) * H
              + yb[pl.ds(s, 16)]) * W + xb[pl.ds(s, 16)]
        lb[pl.ds(s, 16)] = lv
        return 0
    lax.fori_loop(0, CH // 16, init, 0)

    def scan1(k, nb, vb):
        dz = k // 9 - 1
        dy = (k // 3) % 3 - 1
        dx = k % 3 - 1
        dlin = (dz * H + dy) * W + dx

        def body(j, _):
            s = j * 16
            nz = zb[pl.ds(s, 16)] + dz
            ny = yb[pl.ds(s, 16)] + dy
            nx = xb[pl.ds(s, 16)] + dx
            ok = ((nz >= 0) & (nz < D) & (ny >= 0) & (ny < H)
                  & (nx >= 0) & (nx < W))
            spread = (base + s + _iota16()) & 0xFFFF
            nb[pl.ds(s, 16)] = jnp.where(ok, lb[pl.ds(s, 16)] + dlin, spread)
            vb[pl.ds(s, 16)] = jnp.where(ok, 1, 0)
            return 0
        lax.fori_loop(0, CH // 16, body, 0)

    def compact(k, vb, tb, off):
        def center(off):
            def body(j, _):
                s = j * 16
                nidx = tb[pl.ds(s, 16)]
                ok = (vb[pl.ds(s, 16)] > 0) & (nidx >= 0)
                snt = SENT_BASE + ((base + s + _iota16()) & SENT_MASK)
                cvb[pl.ds(s, 16)] = jnp.where(ok, nidx, snt)
                return 0
            lax.fori_loop(0, CH // 16, body, 0)
            pltpu.sync_copy(cvb, cidx_h.at[pl.ds(base, CH)])
            return off

        def taps(off):
            def body(j, off):
                s = j * 16
                nidx = tb[pl.ds(s, 16)]
                li = s + _iota16()
                ok = (vb[pl.ds(s, 16)] > 0) & (nidx >= 0)
                cs1 = plsc.cumsum(jnp.where(ok, 1, 0))
                pos1 = off - 1 + cs1
                plsc.store_scatter(pn, [pos1], nidx, mask=ok)
                plsc.store_scatter(pm, [pos1], li * 32 + k, mask=ok)
                return off + cs1[15]
            return lax.fori_loop(0, CH // 16, body, off)
        return lax.cond(k == 13, center, taps, off)

    def flush(off, flushed):
        nch = off // CHUNK
        flushed = pl.multiple_of(flushed, 8)

        def fl(j, _):
            pltpu.sync_copy(pn.at[pl.ds(j * CHUNK, CHUNK)],
                            pn_h.at[pl.ds(pbase + flushed + j * CHUNK,
                                          CHUNK)])
            pltpu.sync_copy(pm.at[pl.ds(j * CHUNK, CHUNK)],
                            pm_h.at[pl.ds(pbase + flushed + j * CHUNK,
                                          CHUNK)])
            return 0
        lax.fori_loop(0, nch, fl, 0)
        rem = off - nch * CHUNK

        nb = pl.multiple_of(nch * CHUNK, 8)

        def mv(j, _):
            s = j * 16
            pn[pl.ds(s, 16)] = pn[pl.ds(nb + s, 16)]
            pm[pl.ds(s, 16)] = pm[pl.ds(nb + s, 16)]
            return 0
        lax.fori_loop(0, (rem + 15) // 16, mv, 0)
        return rem, flushed + nch * CHUNK

    def maybe_flush(off, flushed):
        return lax.cond(off >= FLUSH_T, flush,
                        lambda o, f: (o, f), off, flushed)

    scan1(0, nb0, vb0)
    pltpu.async_copy(table_h.at[nb0], tb0, gs0)

    def pair_loop(j, carry):
        off, flushed = carry
        k0 = 2 * j + 1
        k1 = 2 * j + 2
        scan1(k0, nb1, vb1)
        pltpu.async_copy(table_h.at[nb1], tb1, gs1)

        pltpu.make_async_copy(table_h.at[nb0], tb0, gs0).wait()
        off = compact(2 * j, vb0, tb0, off)
        off, flushed = maybe_flush(off, flushed)

        scan1(k1, nb0, vb0)
        pltpu.async_copy(table_h.at[nb0], tb0, gs0)

        pltpu.make_async_copy(table_h.at[nb1], tb1, gs1).wait()
        off = compact(k0, vb1, tb1, off)
        off, flushed = maybe_flush(off, flushed)
        return off, flushed
    off, flushed = lax.fori_loop(0, 13, pair_loop, (0, 0))

    pltpu.make_async_copy(table_h.at[nb0], tb0, gs0).wait()
    off = compact(26, vb0, tb0, off)

    cnt = flushed + off
    end = ((off + CHUNK - 1) // CHUNK) * CHUNK

    def padw(j, _):
        pos = off + j * 16 + _iota16()
        snt = SENT_BASE + (pos & SENT_MASK)
        m = pos < end
        plsc.store_scatter(pn, [pos], snt, mask=m)
        plsc.store_scatter(pm, [pos], jnp.zeros((16,), jnp.int32), mask=m)
        return 0
    lax.fori_loop(0, (end - off + 15) // 16, padw, 0)

    flushed8 = pl.multiple_of(flushed, 8)

    def fl2(j, _):
        pltpu.sync_copy(pn.at[pl.ds(j * CHUNK, CHUNK)],
                        pn_h.at[pl.ds(pbase + flushed8 + j * CHUNK, CHUNK)])
        pltpu.sync_copy(pm.at[pl.ds(j * CHUNK, CHUNK)],
                        pm_h.at[pl.ds(pbase + flushed8 + j * CHUNK, CHUNK)])
        return 0
    lax.fori_loop(0, end // CHUNK, fl2, 0)

    cb[pl.ds(0, 16)] = jnp.zeros((16,), jnp.int32) + cnt
    pltpu.sync_copy(cb, cnt_h.at[pl.ds(_wid() * 16, 16)])


def _make_apply(out_n):
  @functools.partial(
    pl.kernel,
    out_type=jax.ShapeDtypeStruct((out_n, C), jnp.float32),
    mesh=_mesh,
    scratch_types=[
        pltpu.VMEM((16,), jnp.int32),
        pltpu.VMEM((NKW * C, C), jnp.float32),
        pltpu.VMEM((CHUNK,), jnp.int32),
        pltpu.VMEM((CHUNK,), jnp.int32),
        pltpu.VMEM((CHUNK, C), jnp.float32),
        pltpu.VMEM((CH, C), jnp.float32),
        pltpu.VMEM((CH,), jnp.int32),
        pltpu.SemaphoreType.DMA,
    ],
    compiler_params=pltpu.CompilerParams(use_tc_tiling_on_sc=False,
                                         needs_layout_passes=False),
  )
  def _apply(pn_h, pm_h, cnt_h, cidx_h, src_h, y13_h, wext_h, out_h,
             cntv, wv, iv, mv, gbuf, acc, civ, sem):
      base = _wid() * CH
      pbase = _wid() * PCAP
      pltpu.sync_copy(wext_h, wv)
      pltpu.sync_copy(cnt_h.at[pl.ds(_wid() * 16, 16)], cntv)
      pltpu.sync_copy(cidx_h.at[pl.ds(base, CH)], civ)
      pltpu.async_copy(y13_h.at[civ], acc, sem).wait()
      cnt = cntv[pl.ds(0, 16)][0]
      nch = (cnt + CHUNK - 1) // CHUNK

      def chunk(ci, _):
          pltpu.sync_copy(pn_h.at[pl.ds(pbase + ci * CHUNK, CHUNK)], iv)
          pltpu.sync_copy(pm_h.at[pl.ds(pbase + ci * CHUNK, CHUNK)], mv)
          pltpu.async_copy(src_h.at[iv], gbuf, sem).wait()

          def group(g, _):
              mvec = mv[pl.ds(g * 16, 16)]
              for p in range(16):
                  m = mvec[p]
                  li = m >> 5
                  kk = (m & 31) * C
                  row = gbuf[g * 16 + p]
                  v = row[0] * wv[kk]
                  for c in range(1, C):
                      v = v + row[c] * wv[kk + c]
                  plsc.addupdate(acc.at[li], v)
              return 0
          lax.fori_loop(0, CHUNK // 16, group, 0)
          return 0
      lax.fori_loop(0, nch, chunk, 0)

      if out_n == NPAD:
          pltpu.sync_copy(acc, out_h.at[pl.ds(base, CH)])
      else:
          last = out_n - (NW - 1) * CH

          @pl.when(base + CH <= out_n)
          def _():
              pltpu.sync_copy(acc, out_h.at[pl.ds(base, CH)])

          @pl.when(base + CH > out_n)
          def _():
              pltpu.sync_copy(acc.at[pl.ds(0, last)],
                              out_h.at[pl.ds(base, last)])
  return _apply


_apply_full = _make_apply(NPAD)
_apply_tail = _make_apply(N)


_MMBLK = 4096


def _mm_body(x_ref, w_ref, o_ref):
    o_ref[...] = jnp.dot(x_ref[...], w_ref[...],
                         preferred_element_type=jnp.float32)


def _center_mm(x, w):
    return pl.pallas_call(
        _mm_body,
        grid=(NPAD // _MMBLK,),
        in_specs=[pl.BlockSpec((_MMBLK, C), lambda i: (i, 0)),
                  pl.BlockSpec((C, C), lambda i: (0, 0))],
        out_specs=pl.BlockSpec((_MMBLK, C), lambda i: (i, 0)),
        out_shape=jax.ShapeDtypeStruct((NPAD, C), jnp.float32),
    )(x, w)


def kernel(features, coors, batch_size, W0, W1):
    off = jnp.asarray(batch_size, jnp.int32) - jnp.int32(B)
    bcol = coors[:, 0]
    zcol = coors[:, 1]
    ycol = coors[:, 2]
    xcol = coors[:, 3]
    lin = ((bcol * D + zcol) * H + ycol) * W + xcol + off
    table = jnp.full((TAB,), -1, jnp.int32).at[lin].max(
        jnp.arange(N, dtype=jnp.int32))

    pad = NPAD - N
    bp = jnp.concatenate([bcol, jnp.zeros((pad,), jnp.int32)])
    zp = jnp.concatenate([zcol, jnp.full((pad,), -10000, jnp.int32)])
    yp = jnp.concatenate([ycol, jnp.full((pad,), -10000, jnp.int32)])
    xp = jnp.concatenate([xcol, jnp.full((pad,), -10000, jnp.int32)])
    featp = jnp.concatenate(
        [features, jnp.zeros((pad, C), jnp.float32)])

    pn, pm, cnt, cidx = _rulebook(bp, zp, yp, xp, table)

    wx0 = jnp.concatenate([W0, -W0[13:14]], axis=0).reshape(NKW * C, C)
    wx1 = jnp.concatenate([W1, -W1[13:14]], axis=0).reshape(NKW * C, C)

    y = _center_mm(featp, W0[13])
    h = _apply_full(pn, pm, cnt, cidx, featp, y, wx0)
    y = _center_mm(h, W1[13])
    return _apply_tail(pn, pm, cnt, cidx, h, y, wx1)
